# baseline (speedup 1.0000x reference)
import jax
import jax.numpy as jnp
from jax import lax
from jax.experimental import pallas as pl
from jax.experimental.pallas import tpu as pltpu
from jax.experimental.pallas import tpu_sc as plsc

NC = 2
NS = 16
NW = NC * NS
CH = 128
K = 4


def _make_gather(n_ids: int, d: int, out_shape, interpret: bool = False):
    assert n_ids % (NW * CH * K * 2) == 0
    g_ids = K * CH
    ng = n_ids // (NW * g_ids)
    b_per_w = ng * g_ids
    mesh = plsc.VectorSubcoreMesh(
        core_axis_name="c", subcore_axis_name="s", num_cores=NC, num_subcores=NS
    )

    def body(idx_hbm, table_hbm, out_hbm, idx_v, rows0, rows1,
             gsem0, gsem1, osem0, osem1):
        rows = (rows0, rows1)
        gsem = (gsem0, gsem1)
        osem = (osem0, osem1)
        wid = lax.axis_index("s") * NC + lax.axis_index("c")
        wbase = wid * b_per_w
        pltpu.sync_copy(idx_hbm.at[wid], idx_v)

        def fire_group(g, p):
            pltpu.async_copy(table_hbm.at[idx_v.at[g]], rows[p], gsem[p])

        def drain_group(p):
            pltpu.make_async_copy(
                out_hbm.at[pl.ds(0, g_ids)], rows[p], gsem[p]
            ).wait()

        fire_group(0, 0)

        @pl.loop(0, ng // 2)
        def _(gl):
            for p in range(2):
                g = gl * 2 + p
                q = 1 - p
                @pl.when(g >= 1)
                def _():
                    pltpu.make_async_copy(
                        out_hbm.at[pl.ds(0, g_ids)],
                        rows[q],
                        osem[q],
                    ).wait()

                @pl.when(g + 1 < ng)
                def _():
                    fire_group(g + 1, q)

                drain_group(p)
                pltpu.async_copy(
                    rows[p],
                    out_hbm.at[pl.ds(wbase + g * g_ids, g_ids)],
                    osem[p],
                )

        lp = (ng - 1) % 2
        pltpu.make_async_copy(
            out_hbm.at[pl.ds(0, g_ids)], rows[lp], osem[lp]
        ).wait()

    return pl.kernel(
        body,
        out_type=jax.ShapeDtypeStruct(out_shape, jnp.float32),
        mesh=mesh,
        scratch_types=(
            pltpu.VMEM((ng, g_ids), jnp.int32),
            pltpu.VMEM((K * CH, d), jnp.float32),
            pltpu.VMEM((K * CH, d), jnp.float32),
            pltpu.SemaphoreType.DMA,
            pltpu.SemaphoreType.DMA,
            pltpu.SemaphoreType.DMA,
            pltpu.SemaphoreType.DMA,
        ),
        compiler_params=pltpu.CompilerParams(use_tc_tiling_on_sc=False),
        interpret=interpret,
    )


def kernel(token_ids, embedding_matrix):
    b, t = token_ids.shape
    n = b * t
    d = embedding_matrix.shape[1]
    idx = token_ids.astype(jnp.int32).reshape(NW, n // (NW * K * CH), K * CH)
    out = _make_gather(n, d, (n, d))(idx, embedding_matrix)
    return out.reshape(b, t, d)


# speedup vs baseline: 1.0004x; 1.0004x over previous
import jax
import jax.numpy as jnp
from jax import lax
from jax.experimental import pallas as pl
from jax.experimental.pallas import tpu as pltpu
from jax.experimental.pallas import tpu_sc as plsc

NC = 2
NS = 16
NW = NC * NS
GRP = 256
NBUF = 4


def _make_gather(n_ids: int, d: int, interpret: bool = False):
    assert n_ids % (NW * GRP * NBUF) == 0
    ng = n_ids // (NW * GRP)
    b_per_w = ng * GRP
    mesh = plsc.VectorSubcoreMesh(
        core_axis_name="c", subcore_axis_name="s", num_cores=NC, num_subcores=NS
    )

    def body(idx_hbm, table_hbm, out_hbm, idx_v, rows0, rows1, rows2, rows3,
             gsem0, gsem1, gsem2, gsem3, osem0, osem1, osem2, osem3):
        rows = (rows0, rows1, rows2, rows3)
        gsem = (gsem0, gsem1, gsem2, gsem3)
        osem = (osem0, osem1, osem2, osem3)
        wid = lax.axis_index("s") * NC + lax.axis_index("c")
        wbase = wid * b_per_w
        pltpu.sync_copy(idx_hbm.at[wid], idx_v)

        def fire_gather(g, b):
            pltpu.async_copy(table_hbm.at[idx_v.at[g]], rows[b], gsem[b])

        for g in range(NBUF - 1):
            fire_gather(g, g)

        @pl.loop(0, ng // NBUF)
        def _(gl):
            for b in range(NBUF):
                g = gl * NBUF + b
                pltpu.make_async_copy(
                    out_hbm.at[pl.ds(0, GRP)], rows[b], gsem[b]
                ).wait()
                pltpu.async_copy(
                    rows[b], out_hbm.at[pl.ds(wbase + g * GRP, GRP)], osem[b]
                )
                bf = (b + NBUF - 1) % NBUF

                @pl.when(g >= 1)
                def _():
                    pltpu.make_async_copy(
                        out_hbm.at[pl.ds(0, GRP)], rows[bf], osem[bf]
                    ).wait()

                @pl.when(g + NBUF - 1 < ng)
                def _():
                    fire_gather(g + NBUF - 1, bf)

        lb = (ng - 1) % NBUF
        pltpu.make_async_copy(
            out_hbm.at[pl.ds(0, GRP)], rows[lb], osem[lb]
        ).wait()

    return pl.kernel(
        body,
        out_type=jax.ShapeDtypeStruct((n_ids, d), jnp.float32),
        mesh=mesh,
        scratch_types=(
            (pltpu.VMEM((ng, GRP), jnp.int32),)
            + tuple(pltpu.VMEM((GRP, d), jnp.float32) for _ in range(NBUF))
            + tuple(pltpu.SemaphoreType.DMA for _ in range(2 * NBUF))
        ),
        compiler_params=pltpu.CompilerParams(use_tc_tiling_on_sc=False),
        interpret=interpret,
    )


def kernel(token_ids, embedding_matrix):
    b, t = token_ids.shape
    n = b * t
    d = embedding_matrix.shape[1]
    idx = token_ids.astype(jnp.int32).reshape(NW, n // (NW * GRP), GRP)
    out = _make_gather(n, d)(idx, embedding_matrix)
    return out.reshape(b, t, d)
